# baseline (speedup 1.0000x reference)
import functools

import jax
import jax.numpy as jnp
import numpy as np
from jax import lax
from jax.experimental import pallas as pl
from jax.experimental.pallas import tpu as pltpu
from jax.experimental.pallas import tpu_sc as plsc

N, L, D = 256, 1025, 96
KEEP = (L - 1) // 2
TOK = KEEP + 1
B = N * TOK
NW = 32
BPW = B // NW
C = 456
NCHUNK = BPW // C


def _build_mask() -> np.ndarray:
    n = N * (L - 1)
    R = [[13, 15, 26, 6], [17, 29, 16, 24]]
    ks = [np.uint32(0), np.uint32(42),
          np.uint32(np.uint32(0) ^ np.uint32(42) ^ np.uint32(0x1BD11BDA))]
    with np.errstate(over="ignore"):
        x0 = np.full(n, ks[0], dtype=np.uint32)
        x1 = np.arange(n, dtype=np.uint32) + ks[1]
        for r in range(5):
            for d in R[r % 2]:
                x0 = x0 + x1
                x1 = (x1 << np.uint32(d)) | (x1 >> np.uint32(32 - d))
                x1 = x0 ^ x1
            x0 = x0 + ks[(r + 1) % 3]
            x1 = x1 + ks[(r + 2) % 3] + np.uint32(r + 1)
        bits = x0 ^ x1
    u = ((bits >> np.uint32(9)) | np.uint32(0x3F800000)).view(np.float32)
    u = np.maximum(u - np.float32(1.0), np.float32(0.0)).reshape(N, L - 1)
    pm = np.argsort(u, axis=1, kind="stable").astype(np.int32) + 1
    pm = np.sort(pm[:, :KEEP], axis=1)
    return np.concatenate([np.zeros((N, 1), np.int32), pm], axis=1)


_MASK = _build_mask()


@functools.partial(
    pl.kernel,
    mesh=plsc.VectorSubcoreMesh(core_axis_name="c", subcore_axis_name="s"),
    compiler_params=pltpu.CompilerParams(use_tc_tiling_on_sc=False),
    out_type=jax.ShapeDtypeStruct((B, D), jnp.float32),
    scratch_types=[
        pltpu.VMEM((BPW,), jnp.int32),
        pltpu.VMEM((C, D), jnp.float32),
        pltpu.SemaphoreType.DMA,
    ],
)
def _gather(table_hbm, gidx_hbm, out_hbm, idx_v, rows_v, sem):
    wid = lax.axis_index("s") * 2 + lax.axis_index("c")
    base = wid * BPW
    pltpu.sync_copy(gidx_hbm.at[pl.ds(base, BPW)], idx_v)
    for c in range(NCHUNK):
        pltpu.async_copy(
            table_hbm.at[idx_v.at[pl.ds(c * C, C)]], rows_v, sem
        ).wait()
        pltpu.sync_copy(rows_v, out_hbm.at[pl.ds(base + c * C, C)])


def kernel(x, force_drop):
    flag = (jnp.asarray(force_drop) != 0).astype(jnp.int32)
    mask = jnp.asarray(_MASK) * flag
    gidx = (mask + jnp.arange(N, dtype=jnp.int32)[:, None] * L).reshape(B)
    out = _gather(x.reshape(N * L, D), gidx)
    return out.reshape(N, TOK, D)


# speedup vs baseline: 1.0121x; 1.0121x over previous
import functools

import jax
import jax.numpy as jnp
import numpy as np
from jax import lax
from jax.experimental import pallas as pl
from jax.experimental.pallas import tpu as pltpu
from jax.experimental.pallas import tpu_sc as plsc

N, L, D = 256, 1025, 96
KEEP = (L - 1) // 2
TOK = KEEP + 1
B = N * TOK
NW = 32
BPW = B // NW
C = 456
NCHUNK = BPW // C


def _build_mask() -> np.ndarray:
    n = N * (L - 1)
    R = [[13, 15, 26, 6], [17, 29, 16, 24]]
    ks = [np.uint32(0), np.uint32(42),
          np.uint32(np.uint32(0) ^ np.uint32(42) ^ np.uint32(0x1BD11BDA))]
    with np.errstate(over="ignore"):
        x0 = np.full(n, ks[0], dtype=np.uint32)
        x1 = np.arange(n, dtype=np.uint32) + ks[1]
        for r in range(5):
            for d in R[r % 2]:
                x0 = x0 + x1
                x1 = (x1 << np.uint32(d)) | (x1 >> np.uint32(32 - d))
                x1 = x0 ^ x1
            x0 = x0 + ks[(r + 1) % 3]
            x1 = x1 + ks[(r + 2) % 3] + np.uint32(r + 1)
        bits = x0 ^ x1
    u = ((bits >> np.uint32(9)) | np.uint32(0x3F800000)).view(np.float32)
    u = np.maximum(u - np.float32(1.0), np.float32(0.0)).reshape(N, L - 1)
    pm = np.argsort(u, axis=1, kind="stable").astype(np.int32) + 1
    pm = np.sort(pm[:, :KEEP], axis=1)
    return np.concatenate([np.zeros((N, 1), np.int32), pm], axis=1)


_MASK = _build_mask()


@functools.partial(
    pl.kernel,
    mesh=plsc.VectorSubcoreMesh(core_axis_name="c", subcore_axis_name="s"),
    compiler_params=pltpu.CompilerParams(use_tc_tiling_on_sc=False),
    out_type=jax.ShapeDtypeStruct((B, D), jnp.float32),
    scratch_types=[
        pltpu.VMEM((BPW,), jnp.int32),
        pltpu.VMEM((2, C, D), jnp.float32),
        pltpu.SemaphoreType.DMA,
        pltpu.SemaphoreType.DMA,
        pltpu.SemaphoreType.DMA,
        pltpu.SemaphoreType.DMA,
    ],
)
def _gather(table_hbm, gidx_hbm, out_hbm, idx_v, rows_v, g0, g1, s0, s1):
    wid = lax.axis_index("s") * 2 + lax.axis_index("c")
    base = wid * BPW
    gsem, ssem = [g0, g1], [s0, s1]
    pltpu.sync_copy(gidx_hbm.at[pl.ds(base, BPW)], idx_v)

    def start_gather(c):
        b = c % 2
        return pltpu.async_copy(
            table_hbm.at[idx_v.at[pl.ds(c * C, C)]], rows_v.at[b], gsem[b]
        )

    def start_scatter(c):
        b = c % 2
        return pltpu.async_copy(
            rows_v.at[b], out_hbm.at[pl.ds(base + c * C, C)], ssem[b]
        )

    gathers = [None] * NCHUNK
    scatters = [None] * NCHUNK
    gathers[0] = start_gather(0)
    for c in range(NCHUNK):
        if c + 1 < NCHUNK:
            if c >= 1:
                scatters[c - 1].wait()
            gathers[c + 1] = start_gather(c + 1)
        gathers[c].wait()
        scatters[c] = start_scatter(c)
    scatters[NCHUNK - 2].wait()
    scatters[NCHUNK - 1].wait()


def kernel(x, force_drop):
    flag = (jnp.asarray(force_drop) != 0).astype(jnp.int32)
    mask = jnp.asarray(_MASK) * flag
    gidx = (mask + jnp.arange(N, dtype=jnp.int32)[:, None] * L).reshape(B)
    out = _gather(x.reshape(N * L, D), gidx)
    return out.reshape(N, TOK, D)
